# Initial kernel scaffold; baseline (speedup 1.0000x reference)
import jax
import jax.numpy as jnp
from jax.experimental import pallas as pl


def kernel(x, edge_index, W1_rel, W1_root, b1, bn_g, bn_b, Wc0, Wc1, Wc2, bc, Ws1_l, bs1, Ws1_r, Ws2_l, bs2, Ws2_r, in_g, in_b, Wf_rel, Wf_root, bf):
    raise NotImplementedError("write your pallas kernel here")


# speedup vs baseline: 8.6633x; 8.6633x over previous
import functools

import jax
import jax.numpy as jnp
from jax import lax
from jax.experimental import pallas as pl
from jax.experimental.pallas import tpu as pltpu
from jax.experimental.pallas import tpu_sc as plsc

N = 10000
E = 320000
NC = 2
NS = 16
NW = NC * NS
EPW = E // NW
CH = 80
NCH = EPW // CH
NP = 10240
RPT = NP // NS

_F32 = jnp.float32


def _mesh():
  return plsc.VectorSubcoreMesh(core_axis_name="c", subcore_axis_name="s",
                                num_cores=NC, num_subcores=NS)


@functools.lru_cache(maxsize=None)
def _make_seg_sum(W):
  out_type = jax.ShapeDtypeStruct((NC, NP, W), _F32)
  scratch = [
      pltpu.VMEM((EPW,), jnp.int32),
      pltpu.VMEM((NCH, CH), jnp.int32),
      pltpu.VMEM((CH, W), _F32),
      pltpu.VMEM_SHARED((NP, W), _F32),
      pltpu.SemaphoreType.DMA,
  ]

  def body(table, src, dst, zrows, out, src_v, dst_v, buf, acc, gsem):
    cid = lax.axis_index("c")
    sid = lax.axis_index("s")
    wid = cid * NS + sid
    r0 = sid * RPT
    pltpu.sync_copy(zrows, acc.at[pl.ds(r0, RPT)])
    pltpu.sync_copy(src.at[pl.ds(wid * EPW, EPW)], src_v)
    pltpu.sync_copy(dst.at[wid], dst_v)
    plsc.subcore_barrier()

    def chunk(c, carry):
      base = pl.multiple_of(c * CH, 8)
      pltpu.async_copy(table.at[src_v.at[pl.ds(base, CH)]], buf, gsem).wait()
      pltpu.sync_copy(buf, acc.at[dst_v.at[c]], add=True)
      return carry

    lax.fori_loop(0, NCH, chunk, 0)
    plsc.subcore_barrier()
    pltpu.sync_copy(acc.at[pl.ds(r0, RPT)], out.at[cid, pl.ds(r0, RPT)])

  return pl.kernel(body, out_type=out_type, mesh=_mesh(),
                   scratch_types=scratch, name=f"seg_sum_w{W}")


@functools.lru_cache(maxsize=None)
def _make_deg():
  out_type = jax.ShapeDtypeStruct((NC, NP, 128), _F32)
  scratch = [
      pltpu.VMEM((NCH, CH), jnp.int32),
      pltpu.VMEM((CH, 128), _F32),
      pltpu.VMEM_SHARED((NP, 128), _F32),
  ]

  def body(dst, ones, zrows, dout, dst_v, ones_v, dacc):
    cid = lax.axis_index("c")
    sid = lax.axis_index("s")
    wid = cid * NS + sid
    r0 = sid * RPT
    pltpu.sync_copy(zrows, dacc.at[pl.ds(r0, RPT)])
    pltpu.sync_copy(ones, ones_v)
    pltpu.sync_copy(dst.at[wid], dst_v)
    plsc.subcore_barrier()

    def chunk(c, carry):
      pltpu.sync_copy(ones_v, dacc.at[dst_v.at[c]], add=True)
      return carry

    lax.fori_loop(0, NCH, chunk, 0)
    plsc.subcore_barrier()
    pltpu.sync_copy(dacc.at[pl.ds(r0, RPT)], dout.at[cid, pl.ds(r0, RPT)])

  return pl.kernel(body, out_type=out_type, mesh=_mesh(),
                   scratch_types=scratch, name="deg_count")


def _seg128(*a):
  return _make_seg_sum(128)(*a)


def _dot(a, b):
  return jnp.dot(a, b, preferred_element_type=_F32,
                 precision=lax.Precision.HIGHEST)


BR = 1000
GRID = N // BR

def _b2(w):
  return pl.BlockSpec((BR, w), lambda i: (i, 0))


def _bp(w):
  return pl.BlockSpec((NC, BR, w), lambda i: (0, i, 0))


def _full(*shape):
  return pl.BlockSpec(shape, lambda i: tuple(0 for _ in shape))


def _tc(body, in_specs, out_specs, out_shapes):
  return pl.pallas_call(
      body, grid=(GRID,),
      in_specs=in_specs, out_specs=out_specs,
      out_shape=[jax.ShapeDtypeStruct(s, _F32) for s in out_shapes],
      compiler_params=pltpu.CompilerParams(
          dimension_semantics=("arbitrary",)))


def _acc_stats(i, h, s_ref, q_ref):
  ps = jnp.sum(h, axis=0, keepdims=True)
  pq = jnp.sum(h * h, axis=0, keepdims=True)

  @pl.when(i == 0)
  def _():
    s_ref[...] = ps
    q_ref[...] = pq

  @pl.when(i > 0)
  def _():
    s_ref[...] += ps
    q_ref[...] += pq


def _norm_apply(h, s_ref, q_ref, g_ref, b_ref):
  mu = s_ref[...] / N
  var = q_ref[...] / N - mu * mu
  return (h - mu) * lax.rsqrt(var + 1e-5) * g_ref[...] + b_ref[...]


def _degprep_body(d_ref, dis_ref, invc_ref):
  deg = d_ref[0, :, :1] + d_ref[1, :, :1]
  dis = jnp.where(deg > 0, lax.rsqrt(jnp.maximum(deg, 1e-12)), 0.0)
  dis_ref[...] = jnp.broadcast_to(dis, (BR, 128))
  invc_ref[...] = jnp.broadcast_to(1.0 / jnp.maximum(deg, 1.0), (BR, 128))


def _degprep(d):
  return _tc(_degprep_body,
             [pl.BlockSpec((NC, BR, 128), lambda i: (0, i, 0))],
             [_b2(128), _b2(128)], [(N, 128), (N, 128)])(d)


def _tc1a_body(p_ref, x_ref, wr_ref, wo_ref, b_ref, h_ref, s_ref, q_ref):
  i = pl.program_id(0)
  h = (_dot(p_ref[0] + p_ref[1], wr_ref[...]) +
       _dot(x_ref[...], wo_ref[...]) + b_ref[...])
  h_ref[...] = h
  _acc_stats(i, h, s_ref, q_ref)


def _tc1b_body(h_ref, s_ref, q_ref, g_ref, b_ref, dis_ref, h2_ref, hs_ref):
  h2 = jnp.maximum(_norm_apply(h_ref[...], s_ref, q_ref, g_ref, b_ref), 0.0)
  h2_ref[...] = h2
  hs_ref[...] = h2 * dis_ref[...]


def _tc2_body(p_ref, dis_ref, tx1_ref, hs2_ref):
  dis = dis_ref[...]
  tx1 = (-dis) * (p_ref[0] + p_ref[1])
  tx1_ref[...] = tx1
  hs2_ref[...] = tx1 * dis


def _tc3_body(p_ref, dis_ref, h2_ref, tx1_ref, w0_ref, w1_ref, w2_ref, b_ref,
              h3_ref):
  h2 = h2_ref[...]
  tx2 = 2.0 * ((-dis_ref[...]) * (p_ref[0] + p_ref[1])) - h2
  h = (_dot(h2, w0_ref[...]) + _dot(tx1_ref[...], w1_ref[...]) +
       _dot(tx2, w2_ref[...]) + b_ref[...])
  h3_ref[...] = jnp.maximum(h, 0.0)


def _sage_a_body(p_ref, invc_ref, h_ref, wl_ref, bl_ref, wr_ref,
                 o_ref, s_ref, q_ref):
  i = pl.program_id(0)
  mean = (p_ref[0] + p_ref[1]) * invc_ref[...]
  o = _dot(mean, wl_ref[...]) + bl_ref[...] + _dot(h_ref[...], wr_ref[...])
  nrm = jnp.sqrt(jnp.sum(o * o, axis=-1, keepdims=True))
  o = o / jnp.maximum(nrm, 1e-12)
  o_ref[...] = o
  _acc_stats(i, o, s_ref, q_ref)


def _sage_b_body(o_ref, s_ref, q_ref, g_ref, b_ref, h_ref):
  h_ref[...] = jnp.maximum(
      _norm_apply(o_ref[...], s_ref, q_ref, g_ref, b_ref), 0.0)


def _tc5b_body(o_ref, s_ref, q_ref, g_ref, b_ref, wfo_ref, bf_ref,
               h7_ref, r_ref):
  h7 = jnp.maximum(_norm_apply(o_ref[...], s_ref, q_ref, g_ref, b_ref), 0.0)
  h7_ref[...] = h7
  r_ref[...] = _dot(h7, wfo_ref[...]) + bf_ref[...]


def _tc6_body(p_ref, wfr_ref, r_ref, o_ref):
  o_ref[...] = _dot(p_ref[0] + p_ref[1], wfr_ref[...]) + r_ref[...]


def _sage_a(p, invc, h, wl, bl, wr):
  return _tc(_sage_a_body,
             [_bp(128), _b2(128), _b2(128), _full(128, 128), _full(1, 128),
              _full(128, 128)],
             [_b2(128), _full(1, 128), _full(1, 128)],
             [(N, 128), (1, 128), (1, 128)])(p, invc, h, wl, bl, wr)


def kernel(x, edge_index, W1_rel, W1_root, b1, bn_g, bn_b, Wc0, Wc1, Wc2, bc,
           Ws1_l, bs1, Ws1_r, Ws2_l, bs2, Ws2_r, in_g, in_b, Wf_rel, Wf_root,
           bf):
  src = edge_index[0]
  dst2d = edge_index[1].reshape(NW, NCH, CH)
  z128 = jnp.zeros((RPT, 128), _F32)
  ones_ch = jnp.ones((CH, 128), _F32)
  b1, bn_g, bn_b, bc, bs1, bs2, in_g, in_b = (
      a.reshape(1, 128) for a in (b1, bn_g, bn_b, bc, bs1, bs2, in_g, in_b))
  bf = bf.reshape(1, 64)

  d = _make_deg()(dst2d, ones_ch, z128)
  dis, invc = _degprep(d)

  p1 = _seg128(x, src, dst2d, z128)
  hpre, s1, q1 = _tc(
      _tc1a_body,
      [_bp(128), _b2(128), _full(128, 128), _full(128, 128), _full(1, 128)],
      [_b2(128), _full(1, 128), _full(1, 128)],
      [(N, 128), (1, 128), (1, 128)])(p1, x, W1_rel, W1_root, b1)
  h2, hs1 = _tc(
      _tc1b_body,
      [_b2(128), _full(1, 128), _full(1, 128), _full(1, 128), _full(1, 128),
       _b2(128)],
      [_b2(128), _b2(128)],
      [(N, 128), (N, 128)])(hpre, s1, q1, bn_g, bn_b, dis)

  p2 = _seg128(hs1, src, dst2d, z128)
  tx1, hs2 = _tc(
      _tc2_body, [_bp(128), _b2(128)], [_b2(128), _b2(128)],
      [(N, 128), (N, 128)])(p2, dis)

  p3 = _seg128(hs2, src, dst2d, z128)
  (h3,) = _tc(
      _tc3_body,
      [_bp(128), _b2(128), _b2(128), _b2(128), _full(128, 128),
       _full(128, 128), _full(128, 128), _full(1, 128)],
      [_b2(128)], [(N, 128)])(p3, dis, h2, tx1, Wc0, Wc1, Wc2, bc)

  p4 = _seg128(h3, src, dst2d, z128)
  o4, s4, q4 = _sage_a(p4, invc, h3, Ws1_l, bs1, Ws1_r)
  (h5,) = _tc(
      _sage_b_body,
      [_b2(128), _full(1, 128), _full(1, 128), _full(1, 128), _full(1, 128)],
      [_b2(128)], [(N, 128)])(o4, s4, q4, in_g, in_b)

  p5 = _seg128(h5, src, dst2d, z128)
  o5, s5, q5 = _sage_a(p5, invc, h5, Ws2_l, bs2, Ws2_r)
  h7, r = _tc(
      _tc5b_body,
      [_b2(128), _full(1, 128), _full(1, 128), _full(1, 128), _full(1, 128),
       _full(128, 64), _full(1, 64)],
      [_b2(128), _b2(64)], [(N, 128), (N, 64)])(o5, s5, q5, in_g, in_b,
                                                Wf_root, bf)

  p6 = _seg128(h7, src, dst2d, z128)
  (out,) = _tc(
      _tc6_body, [_bp(128), _full(128, 64), _b2(64)],
      [_b2(64)], [(N, 64)])(p6, Wf_rel, r)
  return out


# speedup vs baseline: 13.2877x; 1.5338x over previous
import functools

import jax
import jax.numpy as jnp
from jax import lax
from jax.experimental import pallas as pl
from jax.experimental.pallas import tpu as pltpu
from jax.experimental.pallas import tpu_sc as plsc

N = 10000
E = 320000
NC = 2
NS = 16
NW = NC * NS
EPW = E // NW
CH = 80
NCH = EPW // CH
NP = 10240
RPT = NP // NS

_F32 = jnp.float32


def _mesh():
  return plsc.VectorSubcoreMesh(core_axis_name="c", subcore_axis_name="s",
                                num_cores=NC, num_subcores=NS)


@functools.lru_cache(maxsize=None)
def _make_seg_sum(W):
  out_type = jax.ShapeDtypeStruct((NC, NP, W), _F32)
  scratch = [
      pltpu.VMEM((EPW,), jnp.int32),
      pltpu.VMEM((NCH, CH), jnp.int32),
      pltpu.VMEM((CH, W), _F32),
      pltpu.VMEM((CH, W), _F32),
      pltpu.VMEM_SHARED((NP, W), _F32),
      pltpu.SemaphoreType.DMA,
      pltpu.SemaphoreType.DMA,
  ]

  def body(table, src, dst, zrows, out, src_v, dst_v, buf0, buf1, acc,
           gsem0, gsem1):
    cid = lax.axis_index("c")
    sid = lax.axis_index("s")
    wid = cid * NS + sid
    r0 = sid * RPT
    pltpu.sync_copy(zrows, acc.at[pl.ds(r0, RPT)])
    pltpu.sync_copy(src.at[pl.ds(wid * EPW, EPW)], src_v)
    pltpu.sync_copy(dst.at[wid], dst_v)
    plsc.subcore_barrier()

    def gather(c, buf, gsem):
      base = pl.multiple_of(c * CH, 8)
      pltpu.async_copy(table.at[src_v.at[pl.ds(base, CH)]], buf, gsem)

    def gwait(buf, gsem):
      pltpu.make_async_copy(table.at[src_v.at[pl.ds(0, CH)]], buf, gsem).wait()

    gather(0, buf0, gsem0)
    gather(1, buf1, gsem1)

    def pair(g, carry):
      c0 = g * 2
      gwait(buf0, gsem0)
      pltpu.sync_copy(buf0, acc.at[dst_v.at[c0]], add=True)

      @pl.when(c0 + 2 < NCH)
      def _():
        gather(c0 + 2, buf0, gsem0)

      @pl.when(c0 + 1 < NCH)
      def _():
        gwait(buf1, gsem1)
        pltpu.sync_copy(buf1, acc.at[dst_v.at[c0 + 1]], add=True)

        @pl.when(c0 + 3 < NCH)
        def _():
          gather(c0 + 3, buf1, gsem1)

      return carry

    lax.fori_loop(0, (NCH + 1) // 2, pair, 0)
    plsc.subcore_barrier()
    pltpu.sync_copy(acc.at[pl.ds(r0, RPT)], out.at[cid, pl.ds(r0, RPT)])

  return pl.kernel(body, out_type=out_type, mesh=_mesh(),
                   scratch_types=scratch, name=f"seg_sum_w{W}")


@functools.lru_cache(maxsize=None)
def _make_deg():
  out_type = jax.ShapeDtypeStruct((NC, NP, 128), _F32)
  scratch = [
      pltpu.VMEM((NCH, CH), jnp.int32),
      pltpu.VMEM((CH, 128), _F32),
      pltpu.VMEM_SHARED((NP, 128), _F32),
  ]

  def body(dst, ones, zrows, dout, dst_v, ones_v, dacc):
    cid = lax.axis_index("c")
    sid = lax.axis_index("s")
    wid = cid * NS + sid
    r0 = sid * RPT
    pltpu.sync_copy(zrows, dacc.at[pl.ds(r0, RPT)])
    pltpu.sync_copy(ones, ones_v)
    pltpu.sync_copy(dst.at[wid], dst_v)
    plsc.subcore_barrier()

    def chunk(c, carry):
      pltpu.sync_copy(ones_v, dacc.at[dst_v.at[c]], add=True)
      return carry

    lax.fori_loop(0, NCH, chunk, 0)
    plsc.subcore_barrier()
    pltpu.sync_copy(dacc.at[pl.ds(r0, RPT)], dout.at[cid, pl.ds(r0, RPT)])

  return pl.kernel(body, out_type=out_type, mesh=_mesh(),
                   scratch_types=scratch, name="deg_count")


def _seg128(*a):
  return _make_seg_sum(128)(*a)


def _dot(a, b):
  return jnp.dot(a, b, preferred_element_type=_F32,
                 precision=lax.Precision.HIGHEST)


BR = 1000
GRID = N // BR

def _b2(w):
  return pl.BlockSpec((BR, w), lambda i: (i, 0))


def _bp(w):
  return pl.BlockSpec((NC, BR, w), lambda i: (0, i, 0))


def _full(*shape):
  return pl.BlockSpec(shape, lambda i: tuple(0 for _ in shape))


def _tc(body, in_specs, out_specs, out_shapes):
  return pl.pallas_call(
      body, grid=(GRID,),
      in_specs=in_specs, out_specs=out_specs,
      out_shape=[jax.ShapeDtypeStruct(s, _F32) for s in out_shapes],
      compiler_params=pltpu.CompilerParams(
          dimension_semantics=("arbitrary",)))


def _acc_stats(i, h, s_ref, q_ref):
  ps = jnp.sum(h, axis=0, keepdims=True)
  pq = jnp.sum(h * h, axis=0, keepdims=True)

  @pl.when(i == 0)
  def _():
    s_ref[...] = ps
    q_ref[...] = pq

  @pl.when(i > 0)
  def _():
    s_ref[...] += ps
    q_ref[...] += pq


def _norm_apply(h, s_ref, q_ref, g_ref, b_ref):
  mu = s_ref[...] / N
  var = q_ref[...] / N - mu * mu
  return (h - mu) * lax.rsqrt(var + 1e-5) * g_ref[...] + b_ref[...]


def _degprep_body(d_ref, dis_ref, invc_ref):
  deg = d_ref[0, :, :1] + d_ref[1, :, :1]
  dis = jnp.where(deg > 0, lax.rsqrt(jnp.maximum(deg, 1e-12)), 0.0)
  dis_ref[...] = jnp.broadcast_to(dis, (BR, 128))
  invc_ref[...] = jnp.broadcast_to(1.0 / jnp.maximum(deg, 1.0), (BR, 128))


def _degprep(d):
  return _tc(_degprep_body,
             [pl.BlockSpec((NC, BR, 128), lambda i: (0, i, 0))],
             [_b2(128), _b2(128)], [(N, 128), (N, 128)])(d)


def _tc1a_body(p_ref, x_ref, wr_ref, wo_ref, b_ref, h_ref, s_ref, q_ref):
  i = pl.program_id(0)
  h = (_dot(p_ref[0] + p_ref[1], wr_ref[...]) +
       _dot(x_ref[...], wo_ref[...]) + b_ref[...])
  h_ref[...] = h
  _acc_stats(i, h, s_ref, q_ref)


def _tc1b_body(h_ref, s_ref, q_ref, g_ref, b_ref, dis_ref, h2_ref, hs_ref):
  h2 = jnp.maximum(_norm_apply(h_ref[...], s_ref, q_ref, g_ref, b_ref), 0.0)
  h2_ref[...] = h2
  hs_ref[...] = h2 * dis_ref[...]


def _tc2_body(p_ref, dis_ref, tx1_ref, hs2_ref):
  dis = dis_ref[...]
  tx1 = (-dis) * (p_ref[0] + p_ref[1])
  tx1_ref[...] = tx1
  hs2_ref[...] = tx1 * dis


def _tc3_body(p_ref, dis_ref, h2_ref, tx1_ref, w0_ref, w1_ref, w2_ref, b_ref,
              h3_ref):
  h2 = h2_ref[...]
  tx2 = 2.0 * ((-dis_ref[...]) * (p_ref[0] + p_ref[1])) - h2
  h = (_dot(h2, w0_ref[...]) + _dot(tx1_ref[...], w1_ref[...]) +
       _dot(tx2, w2_ref[...]) + b_ref[...])
  h3_ref[...] = jnp.maximum(h, 0.0)


def _sage_a_body(p_ref, invc_ref, h_ref, wl_ref, bl_ref, wr_ref,
                 o_ref, s_ref, q_ref):
  i = pl.program_id(0)
  mean = (p_ref[0] + p_ref[1]) * invc_ref[...]
  o = _dot(mean, wl_ref[...]) + bl_ref[...] + _dot(h_ref[...], wr_ref[...])
  nrm = jnp.sqrt(jnp.sum(o * o, axis=-1, keepdims=True))
  o = o / jnp.maximum(nrm, 1e-12)
  o_ref[...] = o
  _acc_stats(i, o, s_ref, q_ref)


def _sage_b_body(o_ref, s_ref, q_ref, g_ref, b_ref, h_ref):
  h_ref[...] = jnp.maximum(
      _norm_apply(o_ref[...], s_ref, q_ref, g_ref, b_ref), 0.0)


def _tc5b_body(o_ref, s_ref, q_ref, g_ref, b_ref, wfo_ref, bf_ref,
               h7_ref, r_ref):
  h7 = jnp.maximum(_norm_apply(o_ref[...], s_ref, q_ref, g_ref, b_ref), 0.0)
  h7_ref[...] = h7
  r_ref[...] = _dot(h7, wfo_ref[...]) + bf_ref[...]


def _tc6_body(p_ref, wfr_ref, r_ref, o_ref):
  o_ref[...] = _dot(p_ref[0] + p_ref[1], wfr_ref[...]) + r_ref[...]


def _sage_a(p, invc, h, wl, bl, wr):
  return _tc(_sage_a_body,
             [_bp(128), _b2(128), _b2(128), _full(128, 128), _full(1, 128),
              _full(128, 128)],
             [_b2(128), _full(1, 128), _full(1, 128)],
             [(N, 128), (1, 128), (1, 128)])(p, invc, h, wl, bl, wr)


def kernel(x, edge_index, W1_rel, W1_root, b1, bn_g, bn_b, Wc0, Wc1, Wc2, bc,
           Ws1_l, bs1, Ws1_r, Ws2_l, bs2, Ws2_r, in_g, in_b, Wf_rel, Wf_root,
           bf):
  src = edge_index[0]
  dst2d = edge_index[1].reshape(NW, NCH, CH)
  z128 = jnp.zeros((RPT, 128), _F32)
  ones_ch = jnp.ones((CH, 128), _F32)
  b1, bn_g, bn_b, bc, bs1, bs2, in_g, in_b = (
      a.reshape(1, 128) for a in (b1, bn_g, bn_b, bc, bs1, bs2, in_g, in_b))
  bf = bf.reshape(1, 64)

  d = _make_deg()(dst2d, ones_ch, z128)
  dis, invc = _degprep(d)

  p1 = _seg128(x, src, dst2d, z128)
  hpre, s1, q1 = _tc(
      _tc1a_body,
      [_bp(128), _b2(128), _full(128, 128), _full(128, 128), _full(1, 128)],
      [_b2(128), _full(1, 128), _full(1, 128)],
      [(N, 128), (1, 128), (1, 128)])(p1, x, W1_rel, W1_root, b1)
  h2, hs1 = _tc(
      _tc1b_body,
      [_b2(128), _full(1, 128), _full(1, 128), _full(1, 128), _full(1, 128),
       _b2(128)],
      [_b2(128), _b2(128)],
      [(N, 128), (N, 128)])(hpre, s1, q1, bn_g, bn_b, dis)

  p2 = _seg128(hs1, src, dst2d, z128)
  tx1, hs2 = _tc(
      _tc2_body, [_bp(128), _b2(128)], [_b2(128), _b2(128)],
      [(N, 128), (N, 128)])(p2, dis)

  p3 = _seg128(hs2, src, dst2d, z128)
  (h3,) = _tc(
      _tc3_body,
      [_bp(128), _b2(128), _b2(128), _b2(128), _full(128, 128),
       _full(128, 128), _full(128, 128), _full(1, 128)],
      [_b2(128)], [(N, 128)])(p3, dis, h2, tx1, Wc0, Wc1, Wc2, bc)

  p4 = _seg128(h3, src, dst2d, z128)
  o4, s4, q4 = _sage_a(p4, invc, h3, Ws1_l, bs1, Ws1_r)
  (h5,) = _tc(
      _sage_b_body,
      [_b2(128), _full(1, 128), _full(1, 128), _full(1, 128), _full(1, 128)],
      [_b2(128)], [(N, 128)])(o4, s4, q4, in_g, in_b)

  p5 = _seg128(h5, src, dst2d, z128)
  o5, s5, q5 = _sage_a(p5, invc, h5, Ws2_l, bs2, Ws2_r)
  h7, r = _tc(
      _tc5b_body,
      [_b2(128), _full(1, 128), _full(1, 128), _full(1, 128), _full(1, 128),
       _full(128, 64), _full(1, 64)],
      [_b2(128), _b2(64)], [(N, 128), (N, 64)])(o5, s5, q5, in_g, in_b,
                                                Wf_root, bf)

  p6 = _seg128(h7, src, dst2d, z128)
  (out,) = _tc(
      _tc6_body, [_bp(128), _full(128, 64), _b2(64)],
      [_b2(64)], [(N, 64)])(p6, Wf_rel, r)
  return out
